# baseline (speedup 1.0000x reference)
import jax
import jax.numpy as jnp
import numpy as np
from jax import lax
from jax.experimental import pallas as pl
from jax.experimental.pallas import tpu as pltpu
from jax.experimental.pallas import tpu_sc as plsc

N = 10000
E = 320000
D = 128
NC = 2
NS = 16
NH = N // NC
EPT = E // NS
B = 80
NCHUNK = EPT // B
NACC = NH + NS
GRP = 5
CPG = NCHUNK // GRP
NCODE = 60
EPS = 1e-5
RPT = 312
TAILN = NH - NS * RPT


def _sc_agg_body(h_hbm, etab_hbm, src_hbm, dst_hbm, a0_hbm, a1_hbm, a2_hbm,
                 out_hbm,
                 src_v, dst_v, code_v, tmp_v, hrows, erows, acc, sem1, sem2):
    cid = lax.axis_index("c")
    sid = lax.axis_index("s")
    lo = cid * NH
    junk = NH + sid

    def zrow(r, _):
        for c in range(D // 16):
            hrows[r, pl.ds(c * 16, 16)] = jnp.zeros((16,), jnp.float32)
        return 0
    lax.fori_loop(0, B, zrow, 0)
    base = sid * RPT
    for k in range(RPT // B):
        pltpu.sync_copy(hrows, acc.at[pl.ds(base + k * B, B)])
    pltpu.sync_copy(hrows.at[pl.ds(0, RPT % B)],
                    acc.at[pl.ds(base + (RPT // B) * B, RPT % B)])

    @pl.when(sid == 0)
    def _():
        pltpu.sync_copy(hrows.at[pl.ds(0, TAILN + NS)],
                        acc.at[pl.ds(NS * RPT, TAILN + NS)])

    plsc.subcore_barrier()

    for g in range(GRP):
        pltpu.sync_copy(src_hbm.at[sid, g], src_v)
        pltpu.sync_copy(dst_hbm.at[sid, g], dst_v)

        pltpu.sync_copy(a0_hbm.at[sid, g], code_v)
        pltpu.sync_copy(a1_hbm.at[sid, g], tmp_v)

        def _horner(mul):
            def step(r, _):
                for c in range(B // 16):
                    s = pl.ds(c * 16, 16)
                    code_v[r, s] = code_v[r, s] * mul + tmp_v[r, s]
                return 0
            lax.fori_loop(0, CPG, step, 0)

        _horner(6)
        pltpu.sync_copy(a2_hbm.at[sid, g], tmp_v)
        _horner(2)

        def loc(r, _):
            for c in range(B // 16):
                s = pl.ds(c * 16, 16)
                t = dst_v[r, s] - lo
                ok = (t >= 0) & (t < NH)
                dst_v[r, s] = jnp.where(ok, t, junk)
            return 0
        lax.fori_loop(0, CPG, loc, 0)

        def chunk(j, _):
            c1 = pltpu.async_copy(h_hbm.at[src_v.at[j]], hrows, sem1)
            c2 = pltpu.async_copy(etab_hbm.at[code_v.at[j]], erows, sem2)
            c1.wait()
            c2.wait()

            def row(r, _):
                for c in range(D // 16):
                    s = pl.ds(c * 16, 16)
                    hrows[r, s] = jnp.maximum(hrows[r, s] + erows[r, s], 0.0)
                return 0
            lax.fori_loop(0, B, row, 0)

            pltpu.sync_copy(hrows, acc.at[dst_v.at[j]], add=True)
            return 0

        lax.fori_loop(0, CPG, chunk, 0)

    plsc.subcore_barrier()

    pltpu.sync_copy(acc.at[pl.ds(base, RPT)],
                    out_hbm.at[pl.ds(lo + base, RPT)])

    @pl.when(sid == 0)
    def _():
        pltpu.sync_copy(acc.at[pl.ds(NS * RPT, TAILN)],
                        out_hbm.at[pl.ds(lo + NS * RPT, TAILN)])


_sc_agg = pl.kernel(
    _sc_agg_body,
    out_type=jax.ShapeDtypeStruct((N, D), jnp.float32),
    mesh=plsc.VectorSubcoreMesh(core_axis_name="c", subcore_axis_name="s"),
    scratch_types=[
        pltpu.VMEM((CPG, B), jnp.int32),
        pltpu.VMEM((CPG, B), jnp.int32),
        pltpu.VMEM((CPG, B), jnp.int32),
        pltpu.VMEM((CPG, B), jnp.int32),
        pltpu.VMEM((B, D), jnp.float32),
        pltpu.VMEM((B, D), jnp.float32),
        pltpu.VMEM_SHARED((NACC, D), jnp.float32),
        pltpu.SemaphoreType.DMA,
        pltpu.SemaphoreType.DMA,
    ],
)


def _etab_body(s_ref, tcat_ref, o_ref):
    for l in range(3):
        o_ref[l] = jax.lax.dot_general(
            s_ref[...], tcat_ref[l],
            (((1,), (0,)), ((), ())),
            preferred_element_type=jnp.float32,
            precision=jax.lax.Precision.HIGHEST,
        )


_etab_call = pl.pallas_call(
    _etab_body,
    out_shape=jax.ShapeDtypeStruct((3, NCODE, D), jnp.float32),
    in_specs=[pl.BlockSpec(memory_space=pltpu.VMEM)] * 2,
    out_specs=pl.BlockSpec(memory_space=pltpu.VMEM),
)


def _mlp_body(h_ref, p_ref, w1_ref, b1_ref, g1_ref, be1_ref,
              w2_ref, b2_ref, gn_ref, bb_ref, flag_ref, o_ref):
    z = h_ref[...] + p_ref[...]
    t = jax.lax.dot_general(
        z, w1_ref[...], (((1,), (1,)), ((), ())),
        preferred_element_type=jnp.float32,
    ) + b1_ref[...]
    m1 = jnp.mean(t, axis=0, keepdims=True)
    v1 = jnp.mean((t - m1) ** 2, axis=0, keepdims=True)
    t = (t - m1) * jax.lax.rsqrt(v1 + EPS) * g1_ref[...] + be1_ref[...]
    t = jnp.maximum(t, 0.0)
    u = jax.lax.dot_general(
        t, w2_ref[...], (((1,), (1,)), ((), ())),
        preferred_element_type=jnp.float32,
    ) + b2_ref[...]
    m2 = jnp.mean(u, axis=0, keepdims=True)
    v2 = jnp.mean((u - m2) ** 2, axis=0, keepdims=True)
    u = (u - m2) * jax.lax.rsqrt(v2 + EPS) * gn_ref[...] + bb_ref[...]
    u = jnp.where(flag_ref[...] > 0.0, jnp.maximum(u, 0.0), u)
    o_ref[...] = u


_mlp_call = pl.pallas_call(
    _mlp_body,
    out_shape=jax.ShapeDtypeStruct((N, D), jnp.float32),
    in_specs=[pl.BlockSpec(memory_space=pltpu.VMEM)] * 11,
    out_specs=pl.BlockSpec(memory_space=pltpu.VMEM),
)


_r = np.arange(NCODE)
_S = np.zeros((NCODE, 13), np.float32)
_S[_r, _r // 12] = 1.0
_S[_r, 5 + (_r // 2) % 6] = 1.0
_S[_r, 11 + _r % 2] = 1.0


def kernel(x, edge_index, edge_attr, batch, params):
    src = edge_index[0].reshape(NS, GRP, CPG, B)
    dst = edge_index[1].reshape(NS, GRP, CPG, B)
    a0 = edge_attr[:, 0].reshape(NS, GRP, CPG, B)
    a1 = edge_attr[:, 1].reshape(NS, GRP, CPG, B)
    a2 = edge_attr[:, 2].reshape(NS, GRP, CPG, B)

    tcat = jnp.stack([
        jnp.concatenate([p['T0'], p['T1'], p['T2']], axis=0) for p in params
    ])
    etabs = _etab_call(jnp.asarray(_S), tcat)

    stk = lambda k: jnp.stack([p[k] for p in params])
    vec = lambda k: jnp.stack([p[k].reshape(1, D) for p in params])
    xs = (etabs, stk('W1'), vec('b1'), vec('g1'), vec('be1'),
          stk('W2'), vec('b2'), vec('gn'), vec('bb'),
          jnp.array([1.0, 1.0, 0.0], jnp.float32).reshape(3, 1, 1))

    def step(h, layer):
        etab, w1, b1, g1, be1, w2, b2, gn, bb, flag = layer
        agg = _sc_agg(h, etab, src, dst, a0, a1, a2)
        h = _mlp_call(h, agg, w1, b1, g1, be1, w2, b2, gn, bb, flag)
        return h, None

    h, _ = lax.scan(step, x, xs)
    return h
